# Initial kernel scaffold; baseline (speedup 1.0000x reference)
import jax
import jax.numpy as jnp
from jax.experimental import pallas as pl


def kernel(inputs, mask):
    raise NotImplementedError("write your pallas kernel here")


# speedup vs baseline: 16.2739x; 16.2739x over previous
import functools

import jax
import jax.numpy as jnp
from jax import lax
from jax.experimental import pallas as pl
from jax.experimental.pallas import tpu as pltpu
from jax.experimental.pallas import tpu_sc as plsc

_L = 16
_NEG = -3.4e38


def _scalar(v):
  return lax.squeeze(lax.slice(v, (0,), (1,)), (0,))


@functools.lru_cache(maxsize=None)
def _make_sparsemax(R, N):
  info = plsc.get_sparse_core_info()
  NC, NS = info.num_cores, info.num_subcores
  NW = NC * NS
  assert R % (2 * NW) == 0 and N % _L == 0
  RPW = R // NW
  PAIRS = RPW // 2
  NCHUNK = N // _L

  mesh = plsc.VectorSubcoreMesh(core_axis_name="c", subcore_axis_name="s")

  @functools.partial(
      pl.kernel,
      out_type=jax.ShapeDtypeStruct((R, N), jnp.float32),
      mesh=mesh,
      scratch_types=[
          pltpu.VMEM((N,), jnp.float32),
          pltpu.VMEM((N,), jnp.float32),
          pltpu.VMEM((N,), jnp.float32),
          pltpu.VMEM((N,), jnp.float32),
          pltpu.VMEM((N + _L,), jnp.float32),
          pltpu.VMEM((N + _L,), jnp.int32),
          pltpu.SemaphoreType.DMA,
          pltpu.SemaphoreType.DMA,
          pltpu.SemaphoreType.DMA,
          pltpu.SemaphoreType.DMA,
      ],
      compiler_params=pltpu.CompilerParams(needs_layout_passes=False),
  )
  def ker(x_hbm, out_hbm, zbuf0, zbuf1, obuf0, obuf1, vals, idxs,
          sin0, sin1, sout0, sout1):
    cid = lax.axis_index("c")
    sid = lax.axis_index("s")
    base = (sid * NC + cid) * RPW

    zf = jnp.zeros((_L,), jnp.float32)
    zi = jnp.zeros((_L,), jnp.int32)
    iota = lax.iota(jnp.int32, _L)
    negv = jnp.full((_L,), _NEG, jnp.float32)

    def process_row(zbuf, obuf):
      def mx(i, acc):
        obuf[pl.ds(i * _L, _L)] = zf
        return jnp.maximum(acc, zbuf[pl.ds(i * _L, _L)])
      acc = lax.fori_loop(0, NCHUNK, mx, negv)
      thr = jnp.full((_L,), jnp.max(acc), jnp.float32) - 1.0

      def flt(i, carry):
        off, s, iv = carry
        v = zbuf[pl.ds(i * _L, _L)]
        m = v > thr
        dst = off + plsc.cumsum(m.astype(jnp.int32)) - 1
        plsc.store_scatter(vals, [dst], v, mask=m)
        plsc.store_scatter(idxs, [dst], iv, mask=m)
        return (off + plsc.all_reduce_population_count(m),
                s + jnp.where(m, v, zf), iv + _L)
      off, s, _ = lax.fori_loop(0, NCHUNK, flt, (zi, zf, iota))
      c0 = _scalar(off)
      plsc.store_scatter(vals, [off + iota], negv)
      nch = (c0 + (_L - 1)) >> 4
      tau0 = (jnp.full((_L,), jnp.sum(s), jnp.float32) - 1.0) / (
          off.astype(jnp.float32))

      def w_cond(st):
        return jnp.logical_not(st[2])

      def w_body(st):
        tau, c_prev, _ = st
        def ch(i, carry):
          sa, ca = carry
          v = vals[pl.ds(i * _L, _L)]
          m = v > tau
          return (sa + jnp.where(m, v, zf),
                  ca + plsc.all_reduce_population_count(m))
        sa, ca = lax.fori_loop(0, nch, ch, (zf, zi))
        c = _scalar(ca)
        tau_new = (jnp.full((_L,), jnp.sum(sa), jnp.float32) - 1.0) / (
            ca.astype(jnp.float32))
        return (tau_new, c, c == c_prev)

      tau, _, _ = lax.while_loop(w_cond, w_body,
                                 (tau0, c0, jnp.asarray(False)))

      def outp(i, carry):
        v = vals[pl.ds(i * _L, _L)]
        ix = idxs[pl.ds(i * _L, _L)]
        m = v > tau
        plsc.store_scatter(obuf, [ix], v - tau, mask=m)
        return carry
      lax.fori_loop(0, nch, outp, 0)

    def row_pair(j, carry):
      r0 = base + 2 * j
      r1 = r0 + 1
      pltpu.make_async_copy(x_hbm.at[r0], zbuf0, sin0).wait()
      pltpu.make_async_copy(x_hbm.at[r1], zbuf1, sin1).start()

      @pl.when(j > 0)
      def _():
        pltpu.make_async_copy(obuf0, out_hbm.at[r0 - 2], sout0).wait()
      process_row(zbuf0, obuf0)
      pltpu.make_async_copy(obuf0, out_hbm.at[r0], sout0).start()

      pltpu.make_async_copy(x_hbm.at[r1], zbuf1, sin1).wait()

      @pl.when(j < PAIRS - 1)
      def _():
        pltpu.make_async_copy(x_hbm.at[r1 + 1], zbuf0, sin0).start()

      @pl.when(j > 0)
      def _():
        pltpu.make_async_copy(obuf1, out_hbm.at[r1 - 2], sout1).wait()
      process_row(zbuf1, obuf1)
      pltpu.make_async_copy(obuf1, out_hbm.at[r1], sout1).start()
      return carry

    pltpu.make_async_copy(x_hbm.at[base], zbuf0, sin0).start()
    lax.fori_loop(0, PAIRS, row_pair, 0)
    pltpu.make_async_copy(obuf0, out_hbm.at[base + RPW - 2], sout0).wait()
    pltpu.make_async_copy(obuf1, out_hbm.at[base + RPW - 1], sout1).wait()

  return ker


@jax.jit
def _sparsemax2d(x):
  return _make_sparsemax(*x.shape)(x)


def kernel(inputs, mask):
  del mask
  b, q, n = inputs.shape
  return _sparsemax2d(inputs.reshape(b * q, n)).reshape(b, q, n)


# speedup vs baseline: 20.0871x; 1.2343x over previous
import functools

import jax
import jax.numpy as jnp
from jax import lax
from jax.experimental import pallas as pl
from jax.experimental.pallas import tpu as pltpu
from jax.experimental.pallas import tpu_sc as plsc

_L = 16
_NEG = -3.4e38


def _scalar(v):
  return lax.squeeze(lax.slice(v, (0,), (1,)), (0,))


@functools.lru_cache(maxsize=None)
def _make_sparsemax(R, N):
  info = plsc.get_sparse_core_info()
  NC, NS = info.num_cores, info.num_subcores
  NW = NC * NS
  assert R % (2 * NW) == 0 and N % _L == 0
  RPW = R // NW
  PAIRS = RPW // 2
  NCHUNK = N // _L

  mesh = plsc.VectorSubcoreMesh(core_axis_name="c", subcore_axis_name="s")

  @functools.partial(
      pl.kernel,
      out_type=jax.ShapeDtypeStruct((R, N), jnp.float32),
      mesh=mesh,
      scratch_types=[
          pltpu.VMEM((N,), jnp.float32),
          pltpu.VMEM((N,), jnp.float32),
          pltpu.VMEM((N,), jnp.float32),
          pltpu.VMEM((N,), jnp.float32),
          pltpu.VMEM((N + _L,), jnp.float32),
          pltpu.VMEM((N + _L,), jnp.int32),
          pltpu.SemaphoreType.DMA,
          pltpu.SemaphoreType.DMA,
          pltpu.SemaphoreType.DMA,
          pltpu.SemaphoreType.DMA,
      ],
      compiler_params=pltpu.CompilerParams(needs_layout_passes=False),
  )
  def ker(x_hbm, out_hbm, zbuf0, zbuf1, obuf0, obuf1, vals, idxs,
          sin0, sin1, sout0, sout1):
    cid = lax.axis_index("c")
    sid = lax.axis_index("s")
    base = (sid * NC + cid) * RPW

    zf = jnp.zeros((_L,), jnp.float32)
    zi = jnp.zeros((_L,), jnp.int32)
    iota = lax.iota(jnp.int32, _L)
    negv = jnp.full((_L,), _NEG, jnp.float32)

    def process_row(zbuf, obuf):
      def mx(i, acc):
        obuf[pl.ds(i * _L, _L)] = zf
        return jnp.maximum(acc, zbuf[pl.ds(i * _L, _L)])
      acc = lax.fori_loop(0, NCHUNK, mx, negv, unroll=8)
      thr = jnp.full((_L,), jnp.max(acc), jnp.float32) - 1.0

      def flt(i, carry):
        off, s, iv = carry
        v = zbuf[pl.ds(i * _L, _L)]
        m = v > thr
        dst = off + plsc.cumsum(m.astype(jnp.int32)) - 1
        plsc.store_scatter(vals, [dst], v, mask=m)
        plsc.store_scatter(idxs, [dst], iv, mask=m)
        return (off + plsc.all_reduce_population_count(m),
                s + jnp.where(m, v, zf), iv + _L)
      off, s, _ = lax.fori_loop(0, NCHUNK, flt, (zi, zf, iota), unroll=4)
      c0 = _scalar(off)
      plsc.store_scatter(vals, [off + iota], negv)
      nch = (c0 + (_L - 1)) >> 4
      tau0 = (jnp.full((_L,), jnp.sum(s), jnp.float32) - 1.0) / (
          off.astype(jnp.float32))

      def w_cond(st):
        return jnp.logical_not(st[2])

      def w_body(st):
        tau, c_prev, _ = st
        def ch(i, carry):
          sa, ca = carry
          v = vals[pl.ds(i * _L, _L)]
          m = v > tau
          return (sa + jnp.where(m, v, zf),
                  ca + plsc.all_reduce_population_count(m))
        sa, ca = lax.fori_loop(0, nch, ch, (zf, zi))
        c = _scalar(ca)
        tau_new = (jnp.full((_L,), jnp.sum(sa), jnp.float32) - 1.0) / (
            ca.astype(jnp.float32))
        return (tau_new, c, c == c_prev)

      tau, _, _ = lax.while_loop(w_cond, w_body,
                                 (tau0, c0, jnp.asarray(False)))

      def outp(i, carry):
        v = vals[pl.ds(i * _L, _L)]
        ix = idxs[pl.ds(i * _L, _L)]
        m = v > tau
        plsc.store_scatter(obuf, [ix], v - tau, mask=m)
        return carry
      lax.fori_loop(0, nch, outp, 0)

    def row_pair(j, carry):
      r0 = base + 2 * j
      r1 = r0 + 1
      pltpu.make_async_copy(x_hbm.at[r0], zbuf0, sin0).wait()
      pltpu.make_async_copy(x_hbm.at[r1], zbuf1, sin1).start()

      @pl.when(j > 0)
      def _():
        pltpu.make_async_copy(obuf0, out_hbm.at[r0 - 2], sout0).wait()
      process_row(zbuf0, obuf0)
      pltpu.make_async_copy(obuf0, out_hbm.at[r0], sout0).start()

      pltpu.make_async_copy(x_hbm.at[r1], zbuf1, sin1).wait()

      @pl.when(j < PAIRS - 1)
      def _():
        pltpu.make_async_copy(x_hbm.at[r1 + 1], zbuf0, sin0).start()

      @pl.when(j > 0)
      def _():
        pltpu.make_async_copy(obuf1, out_hbm.at[r1 - 2], sout1).wait()
      process_row(zbuf1, obuf1)
      pltpu.make_async_copy(obuf1, out_hbm.at[r1], sout1).start()
      return carry

    pltpu.make_async_copy(x_hbm.at[base], zbuf0, sin0).start()
    lax.fori_loop(0, PAIRS, row_pair, 0)
    pltpu.make_async_copy(obuf0, out_hbm.at[base + RPW - 2], sout0).wait()
    pltpu.make_async_copy(obuf1, out_hbm.at[base + RPW - 1], sout1).wait()

  return ker


@jax.jit
def _sparsemax2d(x):
  return _make_sparsemax(*x.shape)(x)


def kernel(inputs, mask):
  del mask
  b, q, n = inputs.shape
  return _sparsemax2d(inputs.reshape(b * q, n)).reshape(b, q, n)


# speedup vs baseline: 30.8731x; 1.5370x over previous
import functools

import jax
import jax.numpy as jnp
from jax import lax
from jax.experimental import pallas as pl
from jax.experimental.pallas import tpu as pltpu
from jax.experimental.pallas import tpu_sc as plsc

_L = 16
_NEG = -3.4e38


@functools.lru_cache(maxsize=None)
def _make_sparsemax(R, N):
  info = plsc.get_sparse_core_info()
  NC, NS = info.num_cores, info.num_subcores
  NW = NC * NS
  assert R % (2 * NW) == 0 and N % _L == 0
  RPW = R // NW
  PAIRS = RPW // 2
  NCHUNK = N // _L

  mesh = plsc.VectorSubcoreMesh(core_axis_name="c", subcore_axis_name="s")

  @functools.partial(
      pl.kernel,
      out_type=jax.ShapeDtypeStruct((R, N), jnp.float32),
      mesh=mesh,
      scratch_types=[
          pltpu.VMEM((N,), jnp.float32),
          pltpu.VMEM((N,), jnp.float32),
          pltpu.VMEM((N,), jnp.float32),
          pltpu.VMEM((N,), jnp.float32),
          pltpu.VMEM((N,), jnp.int32),
          pltpu.SemaphoreType.DMA,
          pltpu.SemaphoreType.DMA,
          pltpu.SemaphoreType.DMA,
          pltpu.SemaphoreType.DMA,
      ],
      compiler_params=pltpu.CompilerParams(needs_layout_passes=False),
  )
  def ker(x_hbm, out_hbm, zbuf0, zbuf1, obuf0, obuf1, idxs,
          sin0, sin1, sout0, sout1):
    cid = lax.axis_index("c")
    sid = lax.axis_index("s")
    base = (sid * NC + cid) * RPW

    zf = jnp.zeros((_L,), jnp.float32)
    zi = jnp.zeros((_L,), jnp.int32)
    iota = lax.iota(jnp.int32, _L)
    negv = jnp.full((_L,), _NEG, jnp.float32)

    lbase = iota * (N // _L)

    def process_row(zbuf, obuf):
      def mx(i, acc):
        obuf[pl.ds(i * _L, _L)] = zf
        return jnp.maximum(acc, zbuf[pl.ds(i * _L, _L)])
      acc = lax.fori_loop(0, NCHUNK, mx, negv, unroll=8)
      thr = jnp.full((_L,), jnp.max(acc), jnp.float32) - 1.0

      def flt(i, carry):
        cnt, s, iv = carry
        v = zbuf[pl.ds(i * _L, _L)]
        m = v > thr
        plsc.store_scatter(idxs, [lbase + cnt], iv, mask=m)
        return (cnt + m.astype(jnp.int32),
                s + jnp.where(m, v, zf), iv + _L)
      cnt, s, _ = lax.fori_loop(0, NCHUNK, flt, (zi, zf, iota), unroll=4)
      maxc = jnp.max(cnt)
      c0 = jnp.sum(cnt)
      tau0 = (jnp.full((_L,), jnp.sum(s), jnp.float32) - 1.0) / (
          jnp.full((_L,), c0, jnp.int32).astype(jnp.float32))

      def gather_cand(i, lm):
        ix = plsc.load_gather(idxs, [lbase + i])
        return ix, plsc.load_gather(zbuf, [ix], mask=lm)

      def w_cond(st):
        return jnp.logical_not(st[2])

      def w_body(st):
        tau, c_prev, _ = st
        def ch(i, carry):
          sa, ca = carry
          lm = i < cnt
          _, v = gather_cand(i, lm)
          m = jnp.logical_and(v > tau, lm)
          return (sa + jnp.where(m, v, zf), ca + m.astype(jnp.int32))
        sa, ca = lax.fori_loop(0, maxc, ch, (zf, zi))
        c = jnp.sum(ca)
        tau_new = (jnp.full((_L,), jnp.sum(sa), jnp.float32) - 1.0) / (
            jnp.full((_L,), c, jnp.int32).astype(jnp.float32))
        return (tau_new, c, c == c_prev)

      tau, _, _ = lax.while_loop(w_cond, w_body,
                                 (tau0, c0, jnp.asarray(False)))

      def outp(i, carry):
        lm = i < cnt
        ix, v = gather_cand(i, lm)
        m = jnp.logical_and(v > tau, lm)
        plsc.store_scatter(obuf, [ix], v - tau, mask=m)
        return carry
      lax.fori_loop(0, maxc, outp, 0)

    def row_pair(j, carry):
      r0 = base + 2 * j
      r1 = r0 + 1
      pltpu.make_async_copy(x_hbm.at[r0], zbuf0, sin0).wait()
      pltpu.make_async_copy(x_hbm.at[r1], zbuf1, sin1).start()

      @pl.when(j > 0)
      def _():
        pltpu.make_async_copy(obuf0, out_hbm.at[r0 - 2], sout0).wait()
      process_row(zbuf0, obuf0)
      pltpu.make_async_copy(obuf0, out_hbm.at[r0], sout0).start()

      pltpu.make_async_copy(x_hbm.at[r1], zbuf1, sin1).wait()

      @pl.when(j < PAIRS - 1)
      def _():
        pltpu.make_async_copy(x_hbm.at[r1 + 1], zbuf0, sin0).start()

      @pl.when(j > 0)
      def _():
        pltpu.make_async_copy(obuf1, out_hbm.at[r1 - 2], sout1).wait()
      process_row(zbuf1, obuf1)
      pltpu.make_async_copy(obuf1, out_hbm.at[r1], sout1).start()
      return carry

    pltpu.make_async_copy(x_hbm.at[base], zbuf0, sin0).start()
    lax.fori_loop(0, PAIRS, row_pair, 0)
    pltpu.make_async_copy(obuf0, out_hbm.at[base + RPW - 2], sout0).wait()
    pltpu.make_async_copy(obuf1, out_hbm.at[base + RPW - 1], sout1).wait()

  return ker


@jax.jit
def _sparsemax2d(x):
  return _make_sparsemax(*x.shape)(x)


def kernel(inputs, mask):
  del mask
  b, q, n = inputs.shape
  return _sparsemax2d(inputs.reshape(b * q, n)).reshape(b, q, n)
